# Initial kernel scaffold; baseline (speedup 1.0000x reference)
import jax
import jax.numpy as jnp
from jax.experimental import pallas as pl


def kernel(features, mask, Wr1, br1, Wr2, br2, Wn1, bn1, Wn2, bn2, Ws1, bs1, Ws2, bs2):
    raise NotImplementedError("write your pallas kernel here")


# speedup vs baseline: 8.5151x; 8.5151x over previous
import jax
import jax.numpy as jnp
from jax.experimental import pallas as pl
from jax.experimental.pallas import tpu as pltpu


def _leaky(x):
    return jnp.where(x > 0, x, 0.01 * x)


def _gcn_kernel(f_ref, Wr1_ref, br1_ref, Wr2_ref, br2_ref,
                Wn1_ref, bn1_ref, Wn2_ref, bn2_ref,
                Ws1_ref, bs1_ref, Ws2_ref, bs2_ref,
                out_ref, R_ref, RN_ref):
    n = f_ref.shape[0]

    h = jnp.dot(f_ref[...], Wr1_ref[...], preferred_element_type=jnp.float32) + br1_ref[...]
    h = _leaky(h)
    r = jnp.dot(h, Wr2_ref[...], preferred_element_type=jnp.float32) + br2_ref[...]
    r = _leaky(r)
    R_ref[...] = r
    RN_ref[...] = jnp.dot(r, Wn1_ref[...], preferred_element_type=jnp.float32) + bn1_ref[...]

    Ws1 = Ws1_ref[...]
    bs1 = bs1_ref[...]
    Ws2 = Ws2_ref[...]
    bs2 = bs2_ref[...]
    Wn1 = Wn1_ref[...]
    Wn2 = Wn2_ref[...]
    bn2 = bn2_ref[...]

    r_last = R_ref[pl.ds(n - 1, 1), :]
    rn_last = RN_ref[pl.ds(n - 1, 1), :]
    out_ref[pl.ds(n - 1, 1), :] = r_last
    b1 = jnp.maximum(rn_last, 0.0)
    s0 = jnp.maximum(jnp.dot(b1, Wn2, preferred_element_type=jnp.float32) + bn2, 0.0)

    def body(t, s):
        idx = n - 1 - t
        a1 = jnp.maximum(jnp.dot(s, Ws1, preferred_element_type=jnp.float32) + bs1, 0.0)
        a2 = jnp.maximum(jnp.dot(a1, Ws2, preferred_element_type=jnp.float32) + bs2, 0.0)
        out_ref[pl.ds(idx, 1), :] = R_ref[pl.ds(idx, 1), :] + a2
        b1 = jnp.maximum(
            jnp.dot(a2, Wn1, preferred_element_type=jnp.float32) + RN_ref[pl.ds(idx, 1), :], 0.0)
        b2 = jnp.maximum(jnp.dot(b1, Wn2, preferred_element_type=jnp.float32) + bn2, 0.0)
        return s + b2

    jax.lax.fori_loop(1, n, body, s0)


def kernel(features, mask, Wr1, br1, Wr2, br2, Wn1, bn1, Wn2, bn2, Ws1, bs1, Ws2, bs2):
    n = features.shape[0]
    hid = Wr1.shape[1]
    emb = Wr2.shape[1]
    out = pl.pallas_call(
        _gcn_kernel,
        out_shape=jax.ShapeDtypeStruct((n, emb), jnp.float32),
        scratch_shapes=[
            pltpu.VMEM((n, emb), jnp.float32),
            pltpu.VMEM((n, hid), jnp.float32),
        ],
    )(features, Wr1, br1.reshape(1, -1), Wr2, br2.reshape(1, -1),
      Wn1, bn1.reshape(1, -1), Wn2, bn2.reshape(1, -1),
      Ws1, bs1.reshape(1, -1), Ws2, bs2.reshape(1, -1))
    return out
